# Initial kernel scaffold; baseline (speedup 1.0000x reference)
import jax
import jax.numpy as jnp
from jax.experimental import pallas as pl


def kernel(x, edge_index, edge_weight):
    raise NotImplementedError("write your pallas kernel here")


# speedup vs baseline: 4.2417x; 4.2417x over previous
import functools

import jax
import jax.numpy as jnp
from jax import lax
from jax.experimental import pallas as pl
from jax.experimental.pallas import tpu as pltpu
from jax.experimental.pallas import tpu_sc as plsc

N = 10000
E = 320000
D = 128

NC = 2
NS = 16
NW = NC * NS

EPW = E // NW
CHUNK = 80
NCHUNK = EPW // CHUNK
SUPER = 8
NCHUNK_P = 128
NSUP = NCHUNK_P // SUPER
NBUF = 4


def _spmm_sc():
    mesh = plsc.VectorSubcoreMesh(core_axis_name="c", subcore_axis_name="s")

    @functools.partial(
        pl.kernel,
        mesh=mesh,
        out_type=jax.ShapeDtypeStruct((NC, N, D), jnp.float32),
        scratch_types=[
            pltpu.VMEM((2, SUPER, CHUNK), jnp.int32),
            pltpu.VMEM((2, SUPER, CHUNK), jnp.int32),
            pltpu.VMEM((2, SUPER, CHUNK), jnp.float32),
            pltpu.VMEM((NBUF, CHUNK, D), jnp.float32),
            pltpu.VMEM_SHARED((N, D), jnp.float32),
            pltpu.SemaphoreType.DMA,
            pltpu.SemaphoreType.DMA,
            pltpu.SemaphoreType.DMA,
            pltpu.SemaphoreType.DMA,
            pltpu.SemaphoreType.DMA,
            pltpu.SemaphoreType.DMA,
            pltpu.SemaphoreType.DMA,
            pltpu.SemaphoreType.DMA,
            pltpu.SemaphoreType.DMA,
        ],
    )
    def k(x_hbm, col_hbm, row_hbm, w_hbm, zero_hbm, out_hbm,
          col_v, row_v, w_v, rows_v, acc_sh, isem,
          g0, g1, g2, g3, s0, s1, s2, s3):
        cid = lax.axis_index("c")
        sid = lax.axis_index("s")
        wid = sid * NC + cid
        gsems = (g0, g1, g2, g3)
        ssems = (s0, s1, s2, s3)

        @pl.when(sid == 0)
        def _():
            pltpu.sync_copy(zero_hbm, acc_sh)

        def idx_copies(sup, p):
            sl = pl.ds(sup * SUPER, SUPER)
            return (
                pltpu.make_async_copy(col_hbm.at[wid, sl], col_v.at[p], isem),
                pltpu.make_async_copy(row_hbm.at[wid, sl], row_v.at[p], isem),
                pltpu.make_async_copy(w_hbm.at[wid, sl], w_v.at[p], isem),
            )

        def gather(p, j, b):
            return pltpu.make_async_copy(
                x_hbm.at[col_v.at[p, j]], rows_v.at[b], gsems[b])

        def scatter(p, j, b):
            return pltpu.make_async_copy(
                rows_v.at[b], acc_sh.at[row_v.at[p, j]], ssems[b])

        for c in idx_copies(0, 0):
            c.start()
        for c in idx_copies(0, 0):
            c.wait()
        gather(0, 0, 0).start()
        gather(0, 1, 1).start()

        plsc.subcore_barrier()

        def scale(p, j, b):
            def grp(g, c2):
                wvec = w_v[p, j, pl.ds(g * 16, 16)]
                for l in range(16):
                    wl = wvec[l]
                    e = g * 16 + l
                    for jj in range(D // 16):
                        sl = pl.ds(jj * 16, 16)
                        rows_v[b, e, sl] = rows_v[b, e, sl] * wl
                return c2

            lax.fori_loop(0, CHUNK // 16, grp, 0)

        def super_body(s, carry):
            p = lax.rem(s, 2)
            q = 1 - p
            for j in range(SUPER):
                b = j % NBUF
                b2 = (b + 2) % NBUF
                gather(p, j, b).wait()
                scale(p, j, b)
                scatter(p, j, b).start(add=True)
                if j >= 2:
                    scatter(p, j - 2, b2).wait()
                else:
                    @pl.when(s > 0)
                    def _():
                        scatter(q, SUPER - 2 + j, b2).wait()
                if j == 1:
                    @pl.when(s < NSUP - 1)
                    def _():
                        for c in idx_copies_next(s, q):
                            c.start()
                if j < SUPER - 2:
                    gather(p, j + 2, b2).start()
                else:
                    @pl.when(s < NSUP - 1)
                    def _():
                        if j == SUPER - 2:
                            for c in idx_copies_next(s, q):
                                c.wait()
                        gather(q, j - (SUPER - 2), b2).start()
            return carry

        def idx_copies_next(s, q):
            sl = pl.ds((s + 1) * SUPER, SUPER)
            return (
                pltpu.make_async_copy(col_hbm.at[wid, sl], col_v.at[q], isem),
                pltpu.make_async_copy(row_hbm.at[wid, sl], row_v.at[q], isem),
                pltpu.make_async_copy(w_hbm.at[wid, sl], w_v.at[q], isem),
            )

        lax.fori_loop(0, NSUP, super_body, 0)

        scatter(1, SUPER - 2, 2).wait()
        scatter(1, SUPER - 1, 3).wait()

        plsc.subcore_barrier()

        r0 = sid * 624
        pltpu.sync_copy(acc_sh.at[pl.ds(r0, 624)],
                        out_hbm.at[cid, pl.ds(r0, 624)])

        @pl.when(sid == NS - 1)
        def _():
            pltpu.sync_copy(acc_sh.at[pl.ds(16 * 624, N - 16 * 624)],
                            out_hbm.at[cid, pl.ds(16 * 624, N - 16 * 624)])

    return k


def _combine_kernel(a_ref, b_ref, o_ref):
    o_ref[...] = a_ref[...] + b_ref[...]


_BLK = 1000


def _combine(partials):
    grid = (N // _BLK,)
    return pl.pallas_call(
        _combine_kernel,
        grid=grid,
        in_specs=[pl.BlockSpec((_BLK, D), lambda i: (i, 0)),
                  pl.BlockSpec((_BLK, D), lambda i: (i, 0))],
        out_specs=pl.BlockSpec((_BLK, D), lambda i: (i, 0)),
        out_shape=jax.ShapeDtypeStruct((N, D), jnp.float32),
    )(partials[0], partials[1])


def _pad_edges(a):
    a = a.reshape(NW, NCHUNK, CHUNK)
    return jnp.pad(a, ((0, 0), (0, NCHUNK_P - NCHUNK), (0, 0)))


@jax.jit
def kernel(x, edge_index, edge_weight):
    row = _pad_edges(edge_index[0])
    col = _pad_edges(edge_index[1])
    w = _pad_edges(edge_weight)
    zeros = jnp.zeros((N, D), jnp.float32)
    partials = _spmm_sc()(x, col, row, w, zeros)
    return _combine(partials)
